# Initial kernel scaffold; baseline (speedup 1.0000x reference)
import jax
import jax.numpy as jnp
from jax.experimental import pallas as pl


def kernel(x, w_in, species_emb, w1, a1, w2, a2, w_out, b_out, r_intrinsic):
    raise NotImplementedError("write your pallas kernel here")


# speedup vs baseline: 22.9350x; 22.9350x over previous
import functools

import jax
import jax.numpy as jnp
from jax import lax
from jax.experimental import pallas as pl
from jax.experimental.pallas import tpu as pltpu
from jax.experimental.pallas import tpu_sc as plsc

N = 64
D = 16
L = 16
NC = 2
NS = 16
NW = NC * NS
NCHUNK = N // L

_R_EMB = 0
_R_W1 = 64
_R_W2 = 320
_R_AB1 = 576
_R_AB2 = 592
_R_WIN0 = 608
_R_WIN1 = 624
_R_GAM = 640
_R_ALPHA = 644
_R_BETA = 645
_R_WO = 646
_R_WOS = 648
_R_W1T = 664
_R_W2T = 680
_TAB_ROWS = 696

_LN2 = 0.6931471805599453
_SQRT2 = 1.4142135623730951
_LOG_POLY = (
    7.0376836292e-2, -1.1514610310e-1, 1.1676998740e-1, -1.2420140846e-1,
    1.4249322787e-1, -1.6668057665e-1, 2.0000714765e-1, -2.4999993993e-1,
    3.3333331174e-1,
)


def _log_chunk(xc):
    xx = jnp.maximum(xc, jnp.float32(1e-6))
    xi = lax.bitcast_convert_type(xx, jnp.int32)
    ei = lax.shift_right_logical(xi, 23)
    mi = lax.bitwise_or(lax.bitwise_and(xi, 0x007FFFFF), 0x3F800000)
    m = lax.bitcast_convert_type(mi, jnp.float32)
    ef = ei.astype(jnp.float32) - jnp.float32(127.0)
    big = m > jnp.float32(_SQRT2)
    ef = jnp.where(big, ef + jnp.float32(1.0), ef)
    m = jnp.where(big, m * jnp.float32(0.5), m)
    t = m - jnp.float32(1.0)
    p = jnp.float32(_LOG_POLY[0])
    for c in _LOG_POLY[1:]:
        p = p * t + jnp.float32(c)
    z = t * t
    y = t * z * p - jnp.float32(0.5) * z + t
    return y + ef * jnp.float32(_LN2)


def _bf16_round(v):
    vi = lax.bitcast_convert_type(v, jnp.int32)
    odd = lax.bitwise_and(lax.shift_right_logical(vi, 16), 1)
    vi = vi + (odd + 0x7FFF)
    vi = lax.bitwise_and(vi, jnp.int32(-65536))
    return lax.bitcast_convert_type(vi, jnp.float32)


def _dyn_splat(v, idxv):
    return lax.gather(
        v, idxv[:, None],
        lax.GatherDimensionNumbers(offset_dims=(), collapsed_slice_dims=(0,),
                                   start_index_map=(0,)),
        (1,), mode=lax.GatherScatterMode.PROMISE_IN_BOUNDS)


def _top2(sk, idxk):
    big = jnp.float32(1e9)
    mx = jnp.max(jnp.maximum(jnp.maximum(sk[0], sk[1]),
                             jnp.maximum(sk[2], sk[3])))
    cand = [jnp.where(sk[k] == mx, idxk[k], big) for k in range(NCHUNK)]
    i1 = jnp.min(jnp.minimum(jnp.minimum(cand[0], cand[1]),
                             jnp.minimum(cand[2], cand[3])))
    neg = jnp.float32(-1e30)
    sk2 = [jnp.where(idxk[k] == i1, neg, sk[k]) for k in range(NCHUNK)]
    mx2 = jnp.max(jnp.maximum(jnp.maximum(sk2[0], sk2[1]),
                              jnp.maximum(sk2[2], sk2[3])))
    cand2 = [jnp.where(sk2[k] == mx2, idxk[k], big) for k in range(NCHUNK)]
    i2 = jnp.min(jnp.minimum(jnp.minimum(cand2[0], cand2[1]),
                             jnp.minimum(cand2[2], cand2[3])))
    return mx, i1, mx2, i2


def _make_sc_kernel(bt, gpw):
    mesh = plsc.VectorSubcoreMesh(core_axis_name="c", subcore_axis_name="s",
                                  num_cores=NC, num_subcores=NS)

    @functools.partial(
        pl.kernel,
        mesh=mesh,
        compiler_params=pltpu.CompilerParams(needs_layout_passes=False,
                                             use_tc_tiling_on_sc=False),
        out_type=jax.ShapeDtypeStruct((bt, N), jnp.float32),
        scratch_types=[
            pltpu.VMEM((gpw, N), jnp.float32),
            pltpu.VMEM((gpw, N), jnp.float32),
            pltpu.VMEM((_TAB_ROWS, L), jnp.float32),
        ],
    )
    def sc_kernel(x_hbm, tab_hbm, out_hbm, x_v, out_v, tab_v):
        cc = lax.axis_index("c")
        ss = lax.axis_index("s")
        wid = ss * NC + cc
        base = wid * gpw
        pltpu.sync_copy(x_hbm.at[pl.ds(base, gpw)], x_v)
        pltpu.sync_copy(tab_hbm, tab_v)

        iota = lax.broadcasted_iota(jnp.int32, (L,), 0)
        iotaf = iota.astype(jnp.float32)
        idxk = [iotaf + jnp.float32(L * k) for k in range(NCHUNK)]
        zero = jnp.zeros((L,), jnp.float32)

        def hb_row(xb_c, lxb_c, emb_j, j, vsp_prev):
            t = (xb_c * tab_v[_R_WIN0 + j, :]
                 + lxb_c * tab_v[_R_WIN1 + j, :])
            t = t + emb_j
            if vsp_prev is not None:
                t = t + vsp_prev[j]
            return _bf16_round(t)

        def layer(xb, lxb, w_base, wt_base, ab_base, vsp_prev):
            s = [zero, zero, zero, zero]
            for cp in (0, 2):
                hb0 = [hb_row(xb[cp], lxb[cp],
                              tab_v[_R_EMB + j * NCHUNK + cp, :], j,
                              vsp_prev) for j in range(D)]
                hb1 = [hb_row(xb[cp + 1], lxb[cp + 1],
                              tab_v[_R_EMB + j * NCHUNK + cp + 1, :], j,
                              vsp_prev) for j in range(D)]
                for i in range(D):
                    a0 = zero
                    a1 = zero
                    for j in range(D):
                        w = tab_v[w_base + i * D + j, :]
                        a0 = a0 + hb0[j] * w
                        a1 = a1 + hb1[j] * w
                    ab = tab_v[ab_base + i, :]
                    s[cp] = s[cp] + _bf16_round(a0) * ab
                    s[cp + 1] = s[cp + 1] + _bf16_round(a1) * ab
            mx, i1, mx2, i2 = _top2(s, idxk)
            r = jnp.exp(jnp.broadcast_to(mx2 - mx, (L,)))
            den = jnp.float32(1.0) + r
            p1b = _bf16_round(jnp.float32(1.0) / den)
            p2b = _bf16_round(r / den)

            def hw_row(mv):
                cv = lax.shift_right_logical(mv, 4)
                lv = lax.bitwise_and(mv, 15)
                xs = _dyn_splat(jnp.where(cv == 0, xb[0],
                                jnp.where(cv == 1, xb[1],
                                jnp.where(cv == 2, xb[2], xb[3]))), lv)
                lxs = _dyn_splat(jnp.where(cv == 0, lxb[0],
                                 jnp.where(cv == 1, lxb[1],
                                 jnp.where(cv == 2, lxb[2], lxb[3]))), lv)
                acc = zero
                for j in range(D):
                    embs = plsc.load_gather(
                        tab_v,
                        [jnp.broadcast_to(jnp.int32(_R_EMB + j * NCHUNK),
                                          (L,)) + cv, lv])
                    hbs = hb_row(xs, lxs, embs, j, vsp_prev)
                    acc = acc + hbs * tab_v[wt_base + j, :]
                return _bf16_round(acc)

            m1 = jnp.broadcast_to(i1, (L,)).astype(jnp.int32)
            m2 = jnp.broadcast_to(i2, (L,)).astype(jnp.int32)
            return p1b * hw_row(m1) + p2b * hw_row(m2)

        def step(g, carry):
            xk = [x_v[g, pl.ds(L * k, L)] for k in range(NCHUNK)]
            lxk = [_log_chunk(xk[k]) for k in range(NCHUNK)]
            xb = [_bf16_round(xk[k]) for k in range(NCHUNK)]
            lxb = [_bf16_round(lxk[k]) for k in range(NCHUNK)]

            v1 = layer(xb, lxb, _R_W1, _R_W1T, _R_AB1, None)
            v1sp = [_dyn_splat(v1, jnp.broadcast_to(jnp.int32(j), (L,)))
                    for j in range(D)]
            v2 = layer(xb, lxb, _R_W2, _R_W2T, _R_AB2, v1sp)

            wo = tab_v[_R_WO, :]
            dv = jnp.broadcast_to(jnp.sum((v1 + v2) * wo), (L,))
            alv = tab_v[_R_ALPHA, :]
            bev = tab_v[_R_BETA, :]
            for k in range(NCHUNK):
                out_v[g, pl.ds(L * k, L)] = (
                    xb[k] * alv + lxb[k] * bev
                    + tab_v[_R_GAM + k, :] + dv)
            return carry

        lax.fori_loop(0, gpw, step, 0)
        pltpu.sync_copy(out_v, out_hbm.at[pl.ds(base, gpw)])

    return sc_kernel


def kernel(x, w_in, species_emb, w1, a1, w2, a2, w_out, b_out, r_intrinsic):
    B, T, _ = x.shape
    bt = B * T
    gpw = bt // NW

    def bf(a):
        a = a.astype(jnp.float32)
        ai = lax.bitcast_convert_type(a, jnp.int32)
        odd = lax.bitwise_and(lax.shift_right_logical(ai, 16), 1)
        ai = ai + (odd + 0x7FFF)
        ai = lax.bitwise_and(ai, jnp.int32(-65536))
        return lax.bitcast_convert_type(ai, jnp.float32)

    def splat(v):
        return jnp.full((L,), v, jnp.float32)

    emb = species_emb.astype(jnp.float32)
    w0b = bf(w_in[:, 0])
    w1inb = bf(w_in[:, 1])
    wo = w_out[0].astype(jnp.float32)
    hi = lax.Precision.HIGHEST
    gamma0 = jnp.matmul(emb, wo, precision=hi) + b_out[0] + r_intrinsic

    rows = []
    rows.append(emb.T.reshape(N, L))
    rows.append(jnp.repeat(bf(w1).reshape(D * D, 1), L, axis=1))
    rows.append(jnp.repeat(bf(w2).reshape(D * D, 1), L, axis=1))
    rows.append(jnp.repeat(bf(a1[0, D:]).reshape(D, 1), L, axis=1))
    rows.append(jnp.repeat(bf(a2[0, D:]).reshape(D, 1), L, axis=1))
    rows.append(jnp.repeat(w0b.reshape(D, 1), L, axis=1))
    rows.append(jnp.repeat(w1inb.reshape(D, 1), L, axis=1))
    rows.append(gamma0.reshape(NCHUNK, L))
    rows.append(splat(jnp.dot(w0b, wo, precision=hi))[None, :])
    rows.append(splat(jnp.dot(w1inb, wo, precision=hi))[None, :])
    rows.append(wo[None, :])
    rows.append(jnp.zeros((1, L), jnp.float32))
    rows.append(jnp.repeat(wo.reshape(D, 1), L, axis=1))
    rows.append(bf(w1).T)
    rows.append(bf(w2).T)
    tab = jnp.concatenate(rows, axis=0).astype(jnp.float32)

    out2d = _make_sc_kernel(bt, gpw)(x.reshape(bt, N), tab)
    return out2d.reshape(B, T, N)
